# baseline (speedup 1.0000x reference)
import functools

import jax
import jax.numpy as jnp
from jax import lax
from jax.experimental import pallas as pl
from jax.experimental.pallas import tpu as pltpu
from jax.experimental.pallas import tpu_sc as plsc

B = 16384
D = 64
DP = 2 * D
NW = 32
BPW = B // NW
C = 128
NCHUNK = BPW // C


def _sc_gather_body(uidx, midx, eu_mf, em_mf, eu_mlp, em_mlp,
                    o_umf, o_mmf, o_umlp, o_mmlp,
                    uiv, miv, bu_mf, bm_mf, bu_mlp, bm_mlp, sem):
    wid = lax.axis_index("s") * 2 + lax.axis_index("c")
    base = wid * BPW
    for c in range(NCHUNK):
        off = base + c * C
        pltpu.sync_copy(uidx.at[pl.ds(off, C)], uiv)
        pltpu.sync_copy(midx.at[pl.ds(off, C)], miv)
        d0 = pltpu.async_copy(eu_mf.at[uiv], bu_mf, sem)
        d1 = pltpu.async_copy(em_mf.at[miv], bm_mf, sem)
        d2 = pltpu.async_copy(eu_mlp.at[uiv], bu_mlp, sem)
        d3 = pltpu.async_copy(em_mlp.at[miv], bm_mlp, sem)
        d0.wait()
        d1.wait()
        d2.wait()
        d3.wait()
        pltpu.sync_copy(bu_mf, o_umf.at[pl.ds(off, C)])
        pltpu.sync_copy(bm_mf, o_mmf.at[pl.ds(off, C)])
        pltpu.sync_copy(bu_mlp, o_umlp.at[pl.ds(off, C)])
        pltpu.sync_copy(bm_mlp, o_mmlp.at[pl.ds(off, C)])


_pair = jax.ShapeDtypeStruct((B, DP), jnp.float32)
_sc_gather = functools.partial(
    pl.kernel,
    out_type=(_pair, _pair, _pair, _pair),
    mesh=plsc.VectorSubcoreMesh(core_axis_name="c", subcore_axis_name="s"),
    scratch_types=[
        pltpu.VMEM((C,), jnp.int32),
        pltpu.VMEM((C,), jnp.int32),
        pltpu.VMEM((C, DP), jnp.float32),
        pltpu.VMEM((C, DP), jnp.float32),
        pltpu.VMEM((C, DP), jnp.float32),
        pltpu.VMEM((C, DP), jnp.float32),
        pltpu.SemaphoreType.DMA,
    ],
)(_sc_gather_body)


BB = 1024
GRID = B // BB


def _tc_mlp_body(upar, mpar, umf_p, mmf_p, umlp_p, mmlp_p,
                 w1u, w1m, b1, w2, b2, w3, b3, w4, b4,
                 wf_mf, wf_h, bf, out):
    usel = upar[...] > 0
    msel = mpar[...] > 0

    def pick(pair, sel):
        return jnp.where(sel, pair[:, D:], pair[:, :D])

    mf = pick(umf_p[...], usel) * pick(mmf_p[...], msel)
    umlp = pick(umlp_p[...], usel)
    mmlp = pick(mmlp_p[...], msel)
    h = jnp.maximum(
        jnp.dot(umlp, w1u[...], preferred_element_type=jnp.float32)
        + jnp.dot(mmlp, w1m[...], preferred_element_type=jnp.float32)
        + b1[...], 0.0)
    h = jnp.maximum(jnp.dot(h, w2[...], preferred_element_type=jnp.float32) + b2[...], 0.0)
    h = jnp.maximum(jnp.dot(h, w3[...], preferred_element_type=jnp.float32) + b3[...], 0.0)
    h = jnp.maximum(jnp.dot(h, w4[...], preferred_element_type=jnp.float32) + b4[...], 0.0)
    pred = (jnp.sum(mf * wf_mf[...], axis=-1)
            + jnp.sum(h * wf_h[...], axis=-1) + bf[0, 0])
    out[...] = jax.nn.sigmoid(pred)


def _const2d(shape):
    return pl.BlockSpec(shape, lambda i: (0, 0))


def kernel(user_indices, movie_indices, Eu_mf, Em_mf, Eu_mlp, Em_mlp,
           W1, b1, W2, b2, W3, b3, W4, b4, Wf, bf):
    upair_idx = lax.div(user_indices, 2)
    mpair_idx = lax.div(movie_indices, 2)
    ue_mf, me_mf, ue_mlp, me_mlp = _sc_gather(
        upair_idx, mpair_idx,
        Eu_mf.reshape(-1, DP), Em_mf.reshape(-1, DP),
        Eu_mlp.reshape(-1, DP), Em_mlp.reshape(-1, DP))

    par_spec = pl.BlockSpec((BB, 1), lambda i: (i, 0))
    pair_spec = pl.BlockSpec((BB, DP), lambda i: (i, 0))
    out = pl.pallas_call(
        _tc_mlp_body,
        grid=(GRID,),
        in_specs=[
            par_spec, par_spec,
            pair_spec, pair_spec, pair_spec, pair_spec,
            _const2d((D, 128)), _const2d((D, 128)), _const2d((1, 128)),
            _const2d((128, 64)), _const2d((1, 64)),
            _const2d((64, 32)), _const2d((1, 32)),
            _const2d((32, 16)), _const2d((1, 16)),
            _const2d((1, D)), _const2d((1, 16)), _const2d((1, 1)),
        ],
        out_specs=pl.BlockSpec((BB,), lambda i: (i,)),
        out_shape=jax.ShapeDtypeStruct((B,), jnp.float32),
        compiler_params=pltpu.CompilerParams(
            dimension_semantics=("arbitrary",),
        ),
    )(
        lax.rem(user_indices, 2).reshape(B, 1),
        lax.rem(movie_indices, 2).reshape(B, 1),
        ue_mf, me_mf, ue_mlp, me_mlp,
        W1[:D], W1[D:], b1.reshape(1, 128),
        W2, b2.reshape(1, 64),
        W3, b3.reshape(1, 32),
        W4, b4.reshape(1, 16),
        Wf[:D, 0].reshape(1, D), Wf[D:, 0].reshape(1, 16), bf.reshape(1, 1),
    )
    return out


# speedup vs baseline: 1.7699x; 1.7699x over previous
import functools

import jax
import jax.numpy as jnp
from jax import lax
from jax.experimental import pallas as pl
from jax.experimental.pallas import tpu as pltpu
from jax.experimental.pallas import tpu_sc as plsc

B = 16384
D = 64
NW = 32
NU = 1000000
NM = 100000

U_CC = 256
U_SLOTS = 123
U_NFULL = NU // U_CC
U_TAIL_W = NU - U_NFULL * U_CC
U_SHIFT = 12

M_CC = 128
M_SLOTS = 25
M_NFULL = NM // M_CC
M_TAIL_W = NM - M_NFULL * M_CC
M_SHIFT = 9

DUMP = B
SENT = 2**31 - 1

MB = 1056
NBK = 8
BKC = 160
CB = 64
NGV = CB // 16
RING = 3
TAILR = 64
MAXW = 672
NPERM = NW * MAXW


def _phase(idx_ref, t_a, t_b, ta_tail, tb_tail, operm_ref, bmap_ref, wid,
           match_idx, match_b, bcols, bbs, ccols, cb_lin, bstage,
           blk_a, blk_b, tail_a, tail_b, stag, sem_blk, sem_sc,
           *, slots, cc, nfull, shift):
    c0 = wid * (slots * cc)
    c0v = jnp.full((16,), c0, jnp.int32)
    c1v = jnp.full((16,), c0 + slots * cc, jnp.int32)
    iota = lax.iota(jnp.int32, 16)
    zero16 = jnp.zeros((16,), jnp.int32)
    rbase = wid * MAXW

    def init_mi(i, carry):
        match_idx[pl.ds(i * 16, 16)] = jnp.full((16,), SENT, jnp.int32)
        return carry

    lax.fori_loop(0, MB // 16, init_mi, 0)

    def init_bk(i, carry):
        bcols[pl.ds(i * 16, 16)] = jnp.full((16,), SENT, jnp.int32)
        return carry

    lax.fori_loop(0, NBK * BKC // 16, init_bk, 0)

    def init_cc(i, carry):
        ccols[pl.ds(i * 16, 16)] = jnp.zeros((16,), jnp.int32)
        return carry

    lax.fori_loop(0, CB // 16, init_cc, 0)

    def init_bs(i, carry):
        bstage[pl.ds(i * 16, 16)] = jnp.full((16,), DUMP, jnp.int32)
        return carry

    lax.fori_loop(0, MAXW // 16, init_bs, 0)

    def scan_body(j, ptrv):
        v = idx_ref[pl.ds(j * 16, 16)]
        m = (v >= c0v) & (v < c1v)
        cm = plsc.cumsum(jnp.where(m, 1, 0).astype(jnp.int32))
        pos = ptrv + cm - 1
        plsc.store_scatter(match_idx, [pos], v - c0v, mask=m)
        bv = jnp.full((16,), j * 16, jnp.int32) + iota
        plsc.store_scatter(match_b, [pos], bv, mask=m)
        return ptrv + plsc.all_reduce_population_count(m)

    lax.fori_loop(0, B // 16, scan_body, zero16)

    shv = jnp.full((16,), shift, jnp.int32)
    for k in range(NBK):
        kv = jnp.full((16,), k, jnp.int32)
        base = jnp.full((16,), k * BKC, jnp.int32)

        def bpass(j, bptrv):
            mv = match_idx[pl.ds(16 * j, 16)]
            m = lax.shift_right_logical(mv, shv) == kv
            cm = plsc.cumsum(jnp.where(m, 1, 0).astype(jnp.int32))
            pos = jnp.minimum(bptrv + cm - 1,
                              jnp.full((16,), BKC - 1, jnp.int32)) + base
            plsc.store_scatter(bcols, [pos], mv, mask=m)
            bb = match_b[pl.ds(16 * j, 16)]
            plsc.store_scatter(bbs, [pos], bb, mask=m)
            return bptrv + plsc.all_reduce_population_count(m)

        lax.fori_loop(0, MB // 16, bpass, zero16)

    pltpu.sync_copy(ta_tail, tail_a)
    pltpu.sync_copy(tb_tail, tail_b)

    def drain(i, carry2):
        pltpu.make_async_copy(
            stag.at[pl.ds(0, 16)], operm_ref.at[pl.ds(0, 16)], sem_sc).wait()
        return carry2

    def process(p, from_tail, state):
        n3, n2, n1, total = state
        parity = lax.rem(p, 2)
        r = lax.rem(p, RING)

        lo = jnp.full((16,), p * cc, jnp.int32)
        hi = jnp.full((16,), (p + 1) * cc, jnp.int32)

        def cb_init(i, carry2):
            cb_lin[pl.ds(i * 16, 16)] = jnp.full((16,), DUMP, jnp.int32)
            return carry2

        lax.fori_loop(0, CB // 16, cb_init, 0)

        bk = (p * cc) >> shift
        bbase = bk * BKC

        def mscan(j, cptrv):
            mv = bcols[pl.ds(bbase + 16 * j, 16)]
            m = (mv >= lo) & (mv < hi)
            cm = plsc.cumsum(jnp.where(m, 1, 0).astype(jnp.int32))
            pos = jnp.minimum(cptrv + cm - 1,
                              jnp.full((16,), CB - 1, jnp.int32))
            plsc.store_scatter(ccols, [pos], mv - lo, mask=m)
            bv = bbs[pl.ds(bbase + 16 * j, 16)]
            plsc.store_scatter(cb_lin, [pos], bv, mask=m)
            return cptrv + plsc.all_reduce_population_count(m)

        cptrv = lax.fori_loop(0, BKC // 16, mscan, zero16)
        cnt = jnp.minimum(jnp.max(cptrv), CB)
        ngv = (cnt + 15) >> 4
        wbase = jnp.minimum(total, MAXW - CB)

        def cpb(i, carry2):
            bstage[pl.ds(wbase + 16 * i, 16)] = cb_lin[pl.ds(16 * i, 16)]
            return carry2

        lax.fori_loop(0, ngv, cpb, 0)

        lax.fori_loop(0, n3, drain, 0)

        clamp = jnp.full((16,), (TAILR if from_tail else cc) - 1, jnp.int32)
        boff = jnp.full((16,), parity * cc, jnp.int32)
        srow = jnp.full((16,), 0, jnp.int32) + r * CB + iota

        def gbody(jm, carry2):
            cv = jnp.minimum(ccols[pl.ds(16 * jm, 16)], clamp)
            cvp = cv + boff
            rows = srow + 16 * jm
            for d in range(D):
                dv = jnp.full((16,), d, jnp.int32)
                if from_tail:
                    va = plsc.load_gather(tail_a, [cv, dv])
                    vb = plsc.load_gather(tail_b, [cv, dv])
                else:
                    va = plsc.load_gather(blk_a, [dv, cvp])
                    vb = plsc.load_gather(blk_b, [dv, cvp])
                plsc.store_scatter(stag, [rows, dv], va)
                plsc.store_scatter(
                    stag, [rows, jnp.full((16,), D + d, jnp.int32)], vb)
            pltpu.async_copy(
                stag.at[pl.ds(pl.multiple_of(r * CB + 16 * jm, 16), 16)],
                operm_ref.at[pl.ds(pl.multiple_of(
                    rbase + wbase + 16 * jm, 16), 16)], sem_sc)
            return carry2

        lax.fori_loop(0, ngv, gbody, 0)
        return (n2, n1, ngv, wbase + 16 * ngv)

    def start_dma(cid, parity):
        pltpu.make_async_copy(
            t_a.at[:, pl.ds(pl.multiple_of(cid * cc, cc), cc)],
            blk_a.at[:, pl.ds(pl.multiple_of(parity * cc, cc), cc)], sem_blk).start()
        pltpu.make_async_copy(
            t_b.at[:, pl.ds(pl.multiple_of(cid * cc, cc), cc)],
            blk_b.at[:, pl.ds(pl.multiple_of(parity * cc, cc), cc)], sem_blk).start()

    def wait_dma(cid, parity):
        pltpu.make_async_copy(
            t_a.at[:, pl.ds(pl.multiple_of(cid * cc, cc), cc)],
            blk_a.at[:, pl.ds(pl.multiple_of(parity * cc, cc), cc)], sem_blk).wait()
        pltpu.make_async_copy(
            t_b.at[:, pl.ds(pl.multiple_of(cid * cc, cc), cc)],
            blk_b.at[:, pl.ds(pl.multiple_of(parity * cc, cc), cc)], sem_blk).wait()

    cid0 = wid * slots
    start_dma(cid0, 0)

    def chunk_body(p, state):
        cid = cid0 + p
        parity = lax.rem(p, 2)

        @pl.when(cid < nfull)
        def _():
            wait_dma(cid, parity)

        @pl.when((p + 1 < slots) & (cid + 1 < nfull))
        def _():
            start_dma(cid + 1, 1 - parity)

        return lax.cond(
            cid < nfull, lambda: process(p, False, state),
            lambda: lax.cond(cid == nfull,
                             lambda: process(p, True, state),
                             lambda: state))

    zero = jnp.int32(0)
    n3, n2, n1, _tot = lax.fori_loop(
        0, slots, chunk_body, (zero, zero, zero, zero))
    lax.fori_loop(0, n3 + n2 + n1, drain, 0)
    pltpu.sync_copy(bstage, bmap_ref.at[pl.ds(pl.multiple_of(rbase, 32), MAXW)])


def _sc1_body(uidx, midx, tu_mf, tm_mf, tu_mlp, tm_mlp,
              ut_mf, ut_mlp, mt_mf, mt_mlp,
              operm_u, bmap_u, operm_m, bmap_m,
              idx_v, match_idx, match_b, bcols, bbs, ccols, cb_lin, bstage,
              blk_a, blk_b, tail_a, tail_b, stag, sem_blk, sem_sc):
    wid = lax.axis_index("s") * 2 + lax.axis_index("c")
    pltpu.sync_copy(uidx, idx_v)
    _phase(idx_v, tu_mf, tu_mlp, ut_mf, ut_mlp, operm_u, bmap_u, wid,
           match_idx, match_b, bcols, bbs, ccols, cb_lin, bstage,
           blk_a, blk_b, tail_a, tail_b, stag, sem_blk, sem_sc,
           slots=U_SLOTS, cc=U_CC, nfull=U_NFULL, shift=U_SHIFT)
    pltpu.sync_copy(midx, idx_v)
    _phase(idx_v, tm_mf, tm_mlp, mt_mf, mt_mlp, operm_m, bmap_m, wid,
           match_idx, match_b, bcols, bbs, ccols, cb_lin, bstage,
           blk_a, blk_b, tail_a, tail_b, stag, sem_blk, sem_sc,
           slots=M_SLOTS, cc=M_CC, nfull=M_NFULL, shift=M_SHIFT)


_sc_gather = functools.partial(
    pl.kernel,
    out_type=(jax.ShapeDtypeStruct((NPERM, 2 * D), jnp.float32),
              jax.ShapeDtypeStruct((NPERM,), jnp.int32),
              jax.ShapeDtypeStruct((NPERM, 2 * D), jnp.float32),
              jax.ShapeDtypeStruct((NPERM,), jnp.int32)),
    mesh=plsc.VectorSubcoreMesh(core_axis_name="c", subcore_axis_name="s"),
    compiler_params=pltpu.CompilerParams(needs_layout_passes=False),
    scratch_types=[
        pltpu.VMEM((B,), jnp.int32),
        pltpu.VMEM((MB,), jnp.int32),
        pltpu.VMEM((MB,), jnp.int32),
        pltpu.VMEM((NBK * BKC,), jnp.int32),
        pltpu.VMEM((NBK * BKC,), jnp.int32),
        pltpu.VMEM((CB,), jnp.int32),
        pltpu.VMEM((CB,), jnp.int32),
        pltpu.VMEM((MAXW,), jnp.int32),
        pltpu.VMEM((D, 2 * U_CC), jnp.float32),
        pltpu.VMEM((D, 2 * U_CC), jnp.float32),
        pltpu.VMEM((TAILR, D), jnp.float32),
        pltpu.VMEM((TAILR, D), jnp.float32),
        pltpu.VMEM((RING * CB, 2 * D), jnp.float32),
        pltpu.SemaphoreType.DMA,
        pltpu.SemaphoreType.DMA,
    ],
)(_sc1_body)


BR = B // NW


def _sc2_body(operm_u, bmap_u, operm_m, bmap_m, out_u, out_m,
              bmap_v, inv_v, rowv, idxv, sem):
    wid = lax.axis_index("s") * 2 + lax.axis_index("c")
    lo = jnp.full((16,), wid * BR, jnp.int32)
    hi = jnp.full((16,), (wid + 1) * BR, jnp.int32)
    iota = lax.iota(jnp.int32, 16)

    def one(bmap_hbm, operm_hbm, out_hbm):
        pltpu.sync_copy(bmap_hbm, bmap_v)

        def scan(j, carry):
            bm = bmap_v[pl.ds(16 * j, 16)]
            m = (bm >= lo) & (bm < hi)
            rowid = jnp.full((16,), 16 * j, jnp.int32) + iota
            plsc.store_scatter(inv_v, [bm - lo], rowid, mask=m)
            return carry

        lax.fori_loop(0, NPERM // 16, scan, 0)

        for q in range(BR // 128):
            pltpu.async_copy(
                operm_hbm.at[inv_v.at[pl.ds(q * 128, 128)]], rowv, sem).wait()
            pltpu.sync_copy(
                rowv, out_hbm.at[pl.ds(pl.multiple_of(
                    wid * BR + q * 128, 128), 128)])

    one(bmap_u, operm_u, out_u)
    one(bmap_m, operm_m, out_m)


_sc_unperm = functools.partial(
    pl.kernel,
    out_type=(jax.ShapeDtypeStruct((B, 2 * D), jnp.float32),
              jax.ShapeDtypeStruct((B, 2 * D), jnp.float32)),
    mesh=plsc.VectorSubcoreMesh(core_axis_name="c", subcore_axis_name="s"),
    compiler_params=pltpu.CompilerParams(needs_layout_passes=False),
    scratch_types=[
        pltpu.VMEM((NPERM,), jnp.int32),
        pltpu.VMEM((BR,), jnp.int32),
        pltpu.VMEM((128, 2 * D), jnp.float32),
        pltpu.VMEM((128,), jnp.int32),
        pltpu.SemaphoreType.DMA,
    ],
)(_sc2_body)


BB = 1024
GRID = B // BB


def _tc_mlp_body(gu, gm, w1u, w1m, b1, w2, b2, w3, b3, w4, b4,
                 wf_mf, wf_h, bf, out):
    u = gu[...]
    m = gm[...]
    mf = u[:, :D] * m[:, :D]
    h = jnp.maximum(
        jnp.dot(u[:, D:], w1u[...], preferred_element_type=jnp.float32)
        + jnp.dot(m[:, D:], w1m[...], preferred_element_type=jnp.float32)
        + b1[...], 0.0)
    h = jnp.maximum(jnp.dot(h, w2[...], preferred_element_type=jnp.float32) + b2[...], 0.0)
    h = jnp.maximum(jnp.dot(h, w3[...], preferred_element_type=jnp.float32) + b3[...], 0.0)
    h = jnp.maximum(jnp.dot(h, w4[...], preferred_element_type=jnp.float32) + b4[...], 0.0)
    pred = (jnp.sum(mf * wf_mf[...], axis=-1)
            + jnp.sum(h * wf_h[...], axis=-1) + bf[0, 0])
    out[...] = jax.nn.sigmoid(pred)


def _const2d(shape):
    return pl.BlockSpec(shape, lambda i: (0, 0))


def kernel(user_indices, movie_indices, Eu_mf, Em_mf, Eu_mlp, Em_mlp,
           W1, b1, W2, b2, W3, b3, W4, b4, Wf, bf):
    mpad = ((0, TAILR - M_TAIL_W), (0, 0))
    operm_u, bmap_u, operm_m, bmap_m = _sc_gather(
        user_indices, movie_indices,
        Eu_mf.T, Em_mf.T, Eu_mlp.T, Em_mlp.T,
        Eu_mf[U_NFULL * U_CC:], Eu_mlp[U_NFULL * U_CC:],
        jnp.pad(Em_mf[M_NFULL * M_CC:], mpad),
        jnp.pad(Em_mlp[M_NFULL * M_CC:], mpad))
    gath_u, gath_m = _sc_unperm(operm_u, bmap_u, operm_m, bmap_m)

    row_spec = pl.BlockSpec((BB, 2 * D), lambda i: (i, 0))
    out = pl.pallas_call(
        _tc_mlp_body,
        grid=(GRID,),
        in_specs=[
            row_spec, row_spec,
            _const2d((D, 128)), _const2d((D, 128)), _const2d((1, 128)),
            _const2d((128, 64)), _const2d((1, 64)),
            _const2d((64, 32)), _const2d((1, 32)),
            _const2d((32, 16)), _const2d((1, 16)),
            _const2d((1, D)), _const2d((1, 16)), _const2d((1, 1)),
        ],
        out_specs=pl.BlockSpec((BB,), lambda i: (i,)),
        out_shape=jax.ShapeDtypeStruct((B,), jnp.float32),
        compiler_params=pltpu.CompilerParams(
            dimension_semantics=("arbitrary",),
        ),
    )(
        gath_u, gath_m,
        W1[:D], W1[D:], b1.reshape(1, 128),
        W2, b2.reshape(1, 64),
        W3, b3.reshape(1, 32),
        W4, b4.reshape(1, 16),
        Wf[:D, 0].reshape(1, D), Wf[D:, 0].reshape(1, 16), bf.reshape(1, 1),
    )
    return out
